# baseline (speedup 1.0000x reference)
import jax
import jax.numpy as jnp
from jax import lax
from jax.experimental import pallas as pl
from jax.experimental.pallas import tpu as pltpu
from jax.experimental.pallas import tpu_sc as plsc

F = 16
N = 1000000
B = 16384
NW = 32
BPW = B // NW
CHUNK = 128
NCHUNK = BPW // CHUNK
GROUPS = BPW // F


def _body(users, pos_items, neg_items, ue, ie, ub, ib, out,
          idx_u, idx_p, idx_n, ixu, ixp, ixn, tu, tp, tn,
          bu, bp, bn, out_v, sem, bsem):
  wid = lax.axis_index("c") * 16 + lax.axis_index("s")
  base = wid * BPW

  pltpu.sync_copy(users.at[pl.ds(base, BPW)], idx_u)
  pltpu.sync_copy(pos_items.at[pl.ds(base, BPW)], idx_p)
  pltpu.sync_copy(neg_items.at[pl.ds(base, BPW)], idx_n)

  bias_copies = []
  for idx, table, dst in ((idx_u, ub, bu), (idx_p, ib, bp), (idx_n, ib, bn)):
    for c in range(NCHUNK):
      sl = pl.ds(c * CHUNK, CHUNK)
      bias_copies.append(
          pltpu.async_copy(table.at[idx.at[sl]], dst.at[sl], bsem))

  def fill(i, carry):
    sl = pl.ds(i * F, F)
    vu = idx_u[sl]
    vp = idx_p[sl]
    vn = idx_n[sl]
    for f in range(F):
      ixu[f, sl] = vu + f * N
      ixp[f, sl] = vp + f * N
      ixn[f, sl] = vn + f * N
    return carry

  lax.fori_loop(0, BPW // F, fill, 0)

  copies = []
  for ix, table, dst in ((ixu, ue, tu), (ixp, ie, tp), (ixn, ie, tn)):
    for f in range(F):
      for c in range(NCHUNK):
        sl = pl.ds(c * CHUNK, CHUNK)
        copies.append(
            pltpu.async_copy(table.at[ix.at[f].at[sl]],
                             dst.at[f].at[sl], sem))
  for d in bias_copies:
    d.wait()
  for d in copies:
    d.wait()

  for g in range(GROUPS):
    o = pl.ds(g * F, F)
    acc = bu[o] + bp[o] - bn[o]
    for f in range(F):
      acc = acc + tu[f, o] * (tp[f, o] - tn[f, o])
    out_v[o] = acc

  pltpu.sync_copy(out_v, out.at[pl.ds(base, BPW)])


@jax.jit
def kernel(users, pos_items, neg_items, user_embeddings, item_embeddings,
           user_biases, item_biases):
  mesh = plsc.VectorSubcoreMesh(core_axis_name="c", subcore_axis_name="s")
  run = pl.kernel(
      _body,
      out_type=jax.ShapeDtypeStruct((B,), jnp.float32),
      mesh=mesh,
      scratch_types=[
          pltpu.VMEM((BPW,), jnp.int32),
          pltpu.VMEM((BPW,), jnp.int32),
          pltpu.VMEM((BPW,), jnp.int32),
          pltpu.VMEM((F, BPW), jnp.int32),
          pltpu.VMEM((F, BPW), jnp.int32),
          pltpu.VMEM((F, BPW), jnp.int32),
          pltpu.VMEM((F, BPW), jnp.float32),
          pltpu.VMEM((F, BPW), jnp.float32),
          pltpu.VMEM((F, BPW), jnp.float32),
          pltpu.VMEM((BPW,), jnp.float32),
          pltpu.VMEM((BPW,), jnp.float32),
          pltpu.VMEM((BPW,), jnp.float32),
          pltpu.VMEM((BPW,), jnp.float32),
          pltpu.SemaphoreType.DMA,
          pltpu.SemaphoreType.DMA,
      ],
      compiler_params=pltpu.CompilerParams(needs_layout_passes=False),
  )
  return run(users.astype(jnp.int32), pos_items.astype(jnp.int32),
             neg_items.astype(jnp.int32),
             user_embeddings.T.reshape(-1), item_embeddings.T.reshape(-1),
             user_biases.reshape(-1), item_biases.reshape(-1))
